# baseline (speedup 1.0000x reference)
import functools

import jax
import jax.numpy as jnp
from jax import lax
from jax.experimental import pallas as pl
from jax.experimental.pallas import tpu as pltpu
from jax.experimental.pallas import tpu_sc as plsc

M, D = 16384, 2048
NC, NS = 2, 16
NW = NC * NS
ROWS_PER_W = M // NW
CHUNK = 8
NCHUNKS = ROWS_PER_W // CHUNK
NBUF = 6
G = 3

_mesh = plsc.VectorSubcoreMesh(
    core_axis_name="c", subcore_axis_name="s", num_cores=NC, num_subcores=NS
)


@functools.partial(
    pl.kernel,
    out_type=jax.ShapeDtypeStruct((M, D), jnp.float32),
    mesh=_mesh,
    scratch_types=[
        pltpu.VMEM((ROWS_PER_W,), jnp.int32),
        pltpu.VMEM((NBUF, CHUNK, D), jnp.float32),
    ] + [pltpu.SemaphoreType.DMA] * (2 * NBUF),
)
def _gather_rows(idx_hbm, table_hbm, out_hbm, idx_v, rows_v, *sems):
    gsems = sems[:NBUF]
    wsems = sems[NBUF:]
    wid = lax.axis_index("s") * NC + lax.axis_index("c")
    base = wid * ROWS_PER_W

    pltpu.sync_copy(idx_hbm.at[pl.ds(base, ROWS_PER_W)], idx_v)

    def start_gather(c, b):
        pltpu.async_copy(
            table_hbm.at[idx_v.at[pl.ds(c * CHUNK, CHUNK)]],
            rows_v.at[b],
            gsems[b],
        )

    def wait_gather(b):
        pltpu.make_async_copy(
            table_hbm.at[pl.ds(0, CHUNK)], rows_v.at[b], gsems[b]
        ).wait()

    def start_write(c, b):
        pltpu.async_copy(
            rows_v.at[b], out_hbm.at[pl.ds(base + c * CHUNK, CHUNK)], wsems[b]
        )

    def wait_write(b):
        pltpu.make_async_copy(
            rows_v.at[b], out_hbm.at[pl.ds(0, CHUNK)], wsems[b]
        ).wait()

    def step(c, b):
        bn = (b + G) % NBUF
        wait_gather(b)

        @pl.when(c >= NBUF - G)
        def _():
            wait_write(bn)

        @pl.when(c + G < NCHUNKS)
        def _():
            start_gather(c + G, bn)

        start_write(c, b)

    for c in range(G):
        start_gather(c, c)

    _MAIN = (NCHUNKS // NBUF) * NBUF

    @pl.loop(0, _MAIN, step=NBUF)
    def _(c0):
        for i in range(NBUF):
            step(c0 + i, i)

    for c in range(_MAIN, NCHUNKS):
        step(c, c % NBUF)

    for c in range(max(NCHUNKS - (NBUF - G), 0), NCHUNKS):
        wait_write(c % NBUF)


def kernel(indices, state):
    return _gather_rows(indices, state)


# speedup vs baseline: 1.0070x; 1.0070x over previous
import functools

import jax
import jax.numpy as jnp
from jax import lax
from jax.experimental import pallas as pl
from jax.experimental.pallas import tpu as pltpu
from jax.experimental.pallas import tpu_sc as plsc

M, D = 16384, 2048
NC, NS = 2, 16
NW = NC * NS
ROWS_PER_W = M // NW
CHUNK = 8
NCHUNKS = ROWS_PER_W // CHUNK
NBUF = 7
G = 5

_mesh = plsc.VectorSubcoreMesh(
    core_axis_name="c", subcore_axis_name="s", num_cores=NC, num_subcores=NS
)


@functools.partial(
    pl.kernel,
    out_type=jax.ShapeDtypeStruct((M, D), jnp.float32),
    mesh=_mesh,
    scratch_types=[
        pltpu.VMEM((ROWS_PER_W,), jnp.int32),
        pltpu.VMEM((NBUF, CHUNK, D), jnp.float32),
    ] + [pltpu.SemaphoreType.DMA] * (2 * NBUF),
)
def _gather_rows(idx_hbm, table_hbm, out_hbm, idx_v, rows_v, *sems):
    gsems = sems[:NBUF]
    wsems = sems[NBUF:]
    wid = lax.axis_index("s") * NC + lax.axis_index("c")
    base = wid * ROWS_PER_W

    pltpu.sync_copy(idx_hbm.at[pl.ds(base, ROWS_PER_W)], idx_v)

    def start_gather(c, b):
        pltpu.async_copy(
            table_hbm.at[idx_v.at[pl.ds(c * CHUNK, CHUNK)]],
            rows_v.at[b],
            gsems[b],
        )

    def wait_gather(b):
        pltpu.make_async_copy(
            table_hbm.at[pl.ds(0, CHUNK)], rows_v.at[b], gsems[b]
        ).wait()

    def start_write(c, b):
        pltpu.async_copy(
            rows_v.at[b], out_hbm.at[pl.ds(base + c * CHUNK, CHUNK)], wsems[b]
        )

    def wait_write(b):
        pltpu.make_async_copy(
            rows_v.at[b], out_hbm.at[pl.ds(0, CHUNK)], wsems[b]
        ).wait()

    def step(c, b):
        bn = (b + G) % NBUF
        wait_gather(b)

        @pl.when(c >= NBUF - G)
        def _():
            wait_write(bn)

        @pl.when(c + G < NCHUNKS)
        def _():
            start_gather(c + G, bn)

        start_write(c, b)

    for c in range(G):
        start_gather(c, c)

    _MAIN = (NCHUNKS // NBUF) * NBUF

    @pl.loop(0, _MAIN, step=NBUF)
    def _(c0):
        for i in range(NBUF):
            step(c0 + i, i)

    for c in range(_MAIN, NCHUNKS):
        step(c, c % NBUF)

    for c in range(max(NCHUNKS - (NBUF - G), 0), NCHUNKS):
        wait_write(c % NBUF)


def kernel(indices, state):
    return _gather_rows(indices, state)


# speedup vs baseline: 1.0073x; 1.0003x over previous
import functools

import jax
import jax.numpy as jnp
from jax import lax
from jax.experimental import pallas as pl
from jax.experimental.pallas import tpu as pltpu
from jax.experimental.pallas import tpu_sc as plsc

M, D = 16384, 2048
NC, NS = 2, 16
NW = NC * NS
ROWS_PER_W = M // NW
CHUNK = 8
NCHUNKS = ROWS_PER_W // CHUNK
NBUF = 7
G = 4

_mesh = plsc.VectorSubcoreMesh(
    core_axis_name="c", subcore_axis_name="s", num_cores=NC, num_subcores=NS
)


@functools.partial(
    pl.kernel,
    out_type=jax.ShapeDtypeStruct((M, D), jnp.float32),
    mesh=_mesh,
    scratch_types=[
        pltpu.VMEM((ROWS_PER_W,), jnp.int32),
        pltpu.VMEM((NBUF, CHUNK, D), jnp.float32),
    ] + [pltpu.SemaphoreType.DMA] * (2 * NBUF),
)
def _gather_rows(idx_hbm, table_hbm, out_hbm, idx_v, rows_v, *sems):
    gsems = sems[:NBUF]
    wsems = sems[NBUF:]
    wid = lax.axis_index("s") * NC + lax.axis_index("c")
    base = wid * ROWS_PER_W

    pltpu.sync_copy(idx_hbm.at[pl.ds(base, ROWS_PER_W)], idx_v)

    def start_gather(c, b):
        pltpu.async_copy(
            table_hbm.at[idx_v.at[pl.ds(c * CHUNK, CHUNK)]],
            rows_v.at[b],
            gsems[b],
        )

    def wait_gather(b):
        pltpu.make_async_copy(
            table_hbm.at[pl.ds(0, CHUNK)], rows_v.at[b], gsems[b]
        ).wait()

    def start_write(c, b):
        pltpu.async_copy(
            rows_v.at[b], out_hbm.at[pl.ds(base + c * CHUNK, CHUNK)], wsems[b]
        )

    def wait_write(b):
        pltpu.make_async_copy(
            rows_v.at[b], out_hbm.at[pl.ds(0, CHUNK)], wsems[b]
        ).wait()

    def step(c, b):
        bn = (b + G) % NBUF
        wait_gather(b)

        @pl.when(c >= NBUF - G)
        def _():
            wait_write(bn)

        @pl.when(c + G < NCHUNKS)
        def _():
            start_gather(c + G, bn)

        start_write(c, b)

    for c in range(G):
        start_gather(c, c)

    _MAIN = (NCHUNKS // NBUF) * NBUF

    @pl.loop(0, _MAIN, step=NBUF)
    def _(c0):
        for i in range(NBUF):
            step(c0 + i, i)

    for c in range(_MAIN, NCHUNKS):
        step(c, c % NBUF)

    for c in range(max(NCHUNKS - (NBUF - G), 0), NCHUNKS):
        wait_write(c % NBUF)


def kernel(indices, state):
    return _gather_rows(indices, state)
